# baseline (speedup 1.0000x reference)
import functools

import jax
import jax.numpy as jnp
from jax import lax
from jax.experimental import pallas as pl
from jax.experimental.pallas import tpu as pltpu
from jax.experimental.pallas import tpu_sc as plsc

_N_CAND = 8
_ROWS = 2 * 2048
_D = 1024
_NW = 32
_RPW = _ROWS // _NW
_CH = 32
_NCH = _RPW // _CH

_sc_mesh = plsc.VectorSubcoreMesh(core_axis_name="c", subcore_axis_name="s")


@functools.partial(
    pl.kernel,
    out_type=jax.ShapeDtypeStruct((_ROWS, _D), jnp.float32),
    mesh=_sc_mesh,
    scratch_types=[
        pltpu.VMEM((_RPW,), jnp.int32),
        pltpu.VMEM((_CH, _D), jnp.float32),
        pltpu.VMEM((_CH, _D), jnp.float32),
        pltpu.SemaphoreType.DMA,
        pltpu.SemaphoreType.DMA,
    ],
)
def _sc_copy(table_hbm, idx_hbm, out_hbm, idx_v, buf0, buf1, sem0, sem1):
    wid = lax.axis_index("s") * 2 + lax.axis_index("c")
    base = wid * _RPW
    pltpu.sync_copy(idx_hbm.at[pl.ds(base, _RPW)], idx_v)
    bufs = (buf0, buf1)
    sems = (sem0, sem1)
    for ch in range(_NCH):
        b = bufs[ch % 2]
        pltpu.async_copy(
            table_hbm.at[idx_v.at[pl.ds(ch * _CH, _CH)]], b, sems[ch % 2]
        ).wait()
        pltpu.sync_copy(b, out_hbm.at[pl.ds(base + ch * _CH, _CH), :])


def kernel(inputs, binary_gates, alpha, sampled):
    del binary_gates, alpha
    s = jnp.asarray(sampled, dtype=jnp.int32)
    table = inputs.reshape(_N_CAND * _ROWS, _D)
    idx = s * _ROWS + jax.lax.iota(jnp.int32, _ROWS)
    out = _sc_copy(table, idx)
    return out.reshape(2, 2048, _D)


# speedup vs baseline: 1.0737x; 1.0737x over previous
import functools

import jax
import jax.numpy as jnp
from jax import lax
from jax.experimental import pallas as pl
from jax.experimental.pallas import tpu as pltpu
from jax.experimental.pallas import tpu_sc as plsc

_N_CAND = 8
_ROWS = 2 * 2048
_D = 1024
_NW = 32
_RPW = _ROWS // _NW
_CH = 32
_NCH = _RPW // _CH
_NBUF = 3

_sc_mesh = plsc.VectorSubcoreMesh(core_axis_name="c", subcore_axis_name="s")


@functools.partial(
    pl.kernel,
    out_type=jax.ShapeDtypeStruct((_ROWS, _D), jnp.float32),
    mesh=_sc_mesh,
    scratch_types=[
        pltpu.VMEM((_RPW,), jnp.int32),
        [pltpu.VMEM((_CH, _D), jnp.float32) for _ in range(_NBUF)],
        [pltpu.SemaphoreType.DMA for _ in range(_NBUF)],
        [pltpu.SemaphoreType.DMA for _ in range(_NBUF)],
    ],
)
def _sc_copy(table_hbm, idx_hbm, out_hbm, idx_v, bufs, gsems, ssems):
    wid = lax.axis_index("s") * 2 + lax.axis_index("c")
    base = wid * _RPW
    pltpu.sync_copy(idx_hbm.at[pl.ds(base, _RPW)], idx_v)

    def g(ch):
        return pltpu.make_async_copy(
            table_hbm.at[idx_v.at[pl.ds(ch * _CH, _CH)]],
            bufs[ch % _NBUF],
            gsems[ch % _NBUF],
        )

    def s(ch):
        return pltpu.make_async_copy(
            bufs[ch % _NBUF],
            out_hbm.at[pl.ds(base + ch * _CH, _CH), :],
            ssems[ch % _NBUF],
        )

    for ch in range(min(_NBUF, _NCH)):
        g(ch).start()
    for ch in range(_NCH):
        g(ch).wait()
        s(ch).start()
        nxt = ch + _NBUF
        if nxt < _NCH:
            s(ch).wait()
            g(nxt).start()
    for ch in range(max(0, _NCH - _NBUF), _NCH):
        s(ch).wait()


def kernel(inputs, binary_gates, alpha, sampled):
    del binary_gates, alpha
    s = jnp.asarray(sampled, dtype=jnp.int32)
    table = inputs.reshape(_N_CAND * _ROWS, _D)
    idx = s * _ROWS + jax.lax.iota(jnp.int32, _ROWS)
    out = _sc_copy(table, idx)
    return out.reshape(2, 2048, _D)
